# baseline (speedup 1.0000x reference)
import functools
import math

import jax
import jax.numpy as jnp
from jax import lax
from jax.experimental import pallas as pl
from jax.experimental.pallas import tpu as pltpu
from jax.experimental.pallas import tpu_sc as plsc


def _make_sc_kernel(V, D, B, S):
    info = plsc.get_sparse_core_info()
    NC, NS, L = info.num_cores, info.num_subcores, info.num_lanes
    NW = NC * NS
    N = B * S
    assert N % NW == 0 and D % L == 0
    n_per_w = N // NW
    assert S % n_per_w == 0
    w_per_row = S // n_per_w
    T = 16
    NB = 3
    assert n_per_w % T == 0
    n_chunks = n_per_w // T
    d_vecs = D // L
    scale = math.sqrt(float(D))
    n_groups = (n_chunks + NB - 1) // NB

    mesh = plsc.VectorSubcoreMesh(core_axis_name="c", subcore_axis_name="s")

    scratch = [pltpu.VMEM((n_per_w,), jnp.int32)]
    scratch += [pltpu.VMEM((T, D), jnp.float32) for _ in range(2 * NB)]
    scratch += [pltpu.SemaphoreType.DMA for _ in range(3 * NB)]

    @functools.partial(
        pl.kernel,
        mesh=mesh,
        out_type=jax.ShapeDtypeStruct((B, S, D), jnp.float32),
        scratch_types=scratch,
    )
    def k(data_hbm, pos_hbm, idx_hbm, out_hbm, idx_v, *bufs_and_sems):
        A = bufs_and_sems[0:NB]
        Bb = bufs_and_sems[NB:2 * NB]
        sga = bufs_and_sems[2 * NB:3 * NB]
        sgb = bufs_and_sems[3 * NB:4 * NB]
        ss = bufs_and_sems[4 * NB:5 * NB]

        wid = lax.axis_index("s") * NC + lax.axis_index("c")
        row = wid // w_per_row
        col = (wid % w_per_row) * n_per_w
        pltpu.sync_copy(idx_hbm.at[row, pl.ds(col, n_per_w)], idx_v)

        def start_g(c, b):
            idx_c = idx_v.at[pl.ds(c * T, T)]
            pltpu.async_copy(data_hbm.at[idx_c], A[b], sga[b])
            pltpu.async_copy(pos_hbm.at[idx_c], Bb[b], sgb[b])

        def wait_g(b):
            idx0 = idx_v.at[pl.ds(0, T)]
            pltpu.make_async_copy(data_hbm.at[idx0], A[b], sga[b]).wait()
            pltpu.make_async_copy(pos_hbm.at[idx0], Bb[b], sgb[b]).wait()

        def start_s(c, b):
            pltpu.async_copy(A[b], out_hbm.at[row, pl.ds(col + c * T, T)], ss[b])

        def wait_s(b):
            pltpu.make_async_copy(A[b], out_hbm.at[row, pl.ds(col, T)], ss[b]).wait()

        def compute(b):
            @plsc.parallel_loop(0, T, 1, unroll=2)
            def _(t):
                for j in range(d_vecs):
                    sl = pl.ds(j * L, L)
                    A[b][t, sl] = (A[b][t, sl] + Bb[b][t, sl]) * scale

        def phase(c, b):
            bw = (b + 2) % NB

            @pl.when(jnp.logical_and(c >= 1, c + 2 < n_chunks))
            def _():
                wait_s(bw)

            @pl.when(c + 2 < n_chunks)
            def _():
                start_g(c + 2, bw)

            @pl.when(c < n_chunks)
            def _():
                wait_g(b)
                compute(b)
                start_s(c, b)

        start_g(0, 0)
        start_g(1, 1)

        def grp(i, _):
            c0 = NB * i
            phase(c0, 0)
            phase(c0 + 1, 1)
            phase(c0 + 2, 2)
            return 0
        lax.fori_loop(0, n_groups, grp, 0)

        for b in range(NB):
            wait_s(b)

    return k


@functools.lru_cache(maxsize=None)
def _get_kernel(V, D, B, S):
    return _make_sc_kernel(V, D, B, S)


def kernel(x, d_emb, data_table, pos_table):
    B, S = x.shape
    V, D = data_table.shape
    k = _get_kernel(V, D, B, S)
    return k(data_table, pos_table, x.astype(jnp.int32))


# speedup vs baseline: 1.4577x; 1.4577x over previous
import functools
import math

import jax
import jax.numpy as jnp
from jax import lax
from jax.experimental import pallas as pl
from jax.experimental.pallas import tpu as pltpu
from jax.experimental.pallas import tpu_sc as plsc


def _make_sc_kernel(V, D, B, S):
    info = plsc.get_sparse_core_info()
    NC, NS, L = info.num_cores, info.num_subcores, info.num_lanes
    NW = NC * NS
    N = B * S
    assert N % NW == 0 and D % L == 0
    n_per_w = N // NW
    assert S % n_per_w == 0
    w_per_row = S // n_per_w
    T = 16
    NB = 3
    assert n_per_w % T == 0
    n_chunks = n_per_w // T
    d_vecs = D // L
    scale = math.sqrt(float(D))
    n_groups = (n_chunks + NB - 1) // NB

    mesh = plsc.VectorSubcoreMesh(core_axis_name="c", subcore_axis_name="s")

    scratch = [pltpu.VMEM((n_per_w,), jnp.int32)]
    scratch += [pltpu.VMEM((T, D), jnp.float32) for _ in range(2 * NB)]
    scratch += [pltpu.SemaphoreType.DMA for _ in range(3 * NB)]

    @functools.partial(
        pl.kernel,
        mesh=mesh,
        out_type=jax.ShapeDtypeStruct((B, S, D), jnp.float32),
        scratch_types=scratch,
    )
    def k(data_hbm, pos_hbm, idx_hbm, out_hbm, idx_v, *bufs_and_sems):
        A = bufs_and_sems[0:NB]
        Bb = bufs_and_sems[NB:2 * NB]
        sga = bufs_and_sems[2 * NB:3 * NB]
        sgb = bufs_and_sems[3 * NB:4 * NB]
        ss = bufs_and_sems[4 * NB:5 * NB]

        wid = lax.axis_index("s") * NC + lax.axis_index("c")
        row = wid // w_per_row
        col = (wid % w_per_row) * n_per_w
        pltpu.sync_copy(idx_hbm.at[row, pl.ds(col, n_per_w)], idx_v)

        def start_g(c, b):
            idx_c = idx_v.at[pl.ds(c * T, T)]
            pltpu.async_copy(data_hbm.at[idx_c], A[b], sga[b])
            pltpu.async_copy(pos_hbm.at[idx_c], Bb[b], sgb[b])

        def wait_g(b):
            idx0 = idx_v.at[pl.ds(0, T)]
            pltpu.make_async_copy(data_hbm.at[idx0], A[b], sga[b]).wait()
            pltpu.make_async_copy(pos_hbm.at[idx0], Bb[b], sgb[b]).wait()

        def start_s(c, b):
            pltpu.async_copy(A[b], out_hbm.at[row, pl.ds(col + c * T, T)], ss[b])

        def wait_s(b):
            pltpu.make_async_copy(A[b], out_hbm.at[row, pl.ds(col, T)], ss[b]).wait()

        def compute(b):
            def body(t, _):
                for j in range(d_vecs):
                    sl = pl.ds(j * L, L)
                    A[b][t, sl] = (A[b][t, sl] + Bb[b][t, sl]) * scale
                return 0
            lax.fori_loop(0, T, body, 0)

        def phase(c, b):
            bw = (b + 2) % NB

            @pl.when(jnp.logical_and(c >= 1, c + 2 < n_chunks))
            def _():
                wait_s(bw)

            @pl.when(c + 2 < n_chunks)
            def _():
                start_g(c + 2, bw)

            @pl.when(c < n_chunks)
            def _():
                wait_g(b)
                compute(b)
                start_s(c, b)

        start_g(0, 0)
        start_g(1, 1)

        def grp(i, _):
            c0 = NB * i
            phase(c0, 0)
            phase(c0 + 1, 1)
            phase(c0 + 2, 2)
            return 0
        lax.fori_loop(0, n_groups, grp, 0)

        for b in range(NB):
            wait_s(b)

    return k


@functools.lru_cache(maxsize=None)
def _get_kernel(V, D, B, S):
    return _make_sc_kernel(V, D, B, S)


def kernel(x, d_emb, data_table, pos_table):
    B, S = x.shape
    V, D = data_table.shape
    k = _get_kernel(V, D, B, S)
    return k(data_table, pos_table, x.astype(jnp.int32))


# speedup vs baseline: 1.6654x; 1.1425x over previous
import functools
import math

import jax
import jax.numpy as jnp
from jax import lax
from jax.experimental import pallas as pl
from jax.experimental.pallas import tpu as pltpu
from jax.experimental.pallas import tpu_sc as plsc


def _make_sc_kernel(V, D, B, S):
    info = plsc.get_sparse_core_info()
    NC, NS, L = info.num_cores, info.num_subcores, info.num_lanes
    NW = NC * NS
    N = B * S
    assert N % NW == 0 and D % L == 0
    n_per_w = N // NW
    assert S % n_per_w == 0
    w_per_row = S // n_per_w
    T = 8
    NB = 4
    LA = NB - 1
    assert n_per_w % T == 0
    n_chunks = n_per_w // T
    d_vecs = D // L
    scale = math.sqrt(float(D))
    n_groups = (n_chunks + NB - 1) // NB

    mesh = plsc.VectorSubcoreMesh(core_axis_name="c", subcore_axis_name="s")

    scratch = [pltpu.VMEM((n_per_w,), jnp.int32)]
    scratch += [pltpu.VMEM((T, D), jnp.float32) for _ in range(2 * NB)]
    scratch += [pltpu.SemaphoreType.DMA for _ in range(3 * NB)]

    @functools.partial(
        pl.kernel,
        mesh=mesh,
        out_type=jax.ShapeDtypeStruct((B, S, D), jnp.float32),
        scratch_types=scratch,
    )
    def k(data_hbm, pos_hbm, idx_hbm, out_hbm, idx_v, *bufs_and_sems):
        A = bufs_and_sems[0:NB]
        Bb = bufs_and_sems[NB:2 * NB]
        sga = bufs_and_sems[2 * NB:3 * NB]
        sgb = bufs_and_sems[3 * NB:4 * NB]
        ss = bufs_and_sems[4 * NB:5 * NB]

        wid = lax.axis_index("s") * NC + lax.axis_index("c")
        row = wid // w_per_row
        col = (wid % w_per_row) * n_per_w
        pltpu.sync_copy(idx_hbm.at[row, pl.ds(col, n_per_w)], idx_v)

        def start_g(c, b):
            idx_c = idx_v.at[pl.ds(c * T, T)]
            pltpu.async_copy(data_hbm.at[idx_c], A[b], sga[b])
            pltpu.async_copy(pos_hbm.at[idx_c], Bb[b], sgb[b])

        def wait_g(b):
            idx0 = idx_v.at[pl.ds(0, T)]
            pltpu.make_async_copy(data_hbm.at[idx0], A[b], sga[b]).wait()
            pltpu.make_async_copy(pos_hbm.at[idx0], Bb[b], sgb[b]).wait()

        def start_s(c, b):
            pltpu.async_copy(A[b], out_hbm.at[row, pl.ds(col + c * T, T)], ss[b])

        def wait_s(b):
            pltpu.make_async_copy(A[b], out_hbm.at[row, pl.ds(col, T)], ss[b]).wait()

        def compute(b):
            def body(t, _):
                for j in range(d_vecs):
                    sl = pl.ds(j * L, L)
                    A[b][t, sl] = (A[b][t, sl] + Bb[b][t, sl]) * scale
                return 0
            lax.fori_loop(0, T, body, 0)

        def phase(c, b):
            bw = (b + LA) % NB

            @pl.when(jnp.logical_and(c >= 1, c + LA < n_chunks))
            def _():
                wait_s(bw)

            @pl.when(c + LA < n_chunks)
            def _():
                start_g(c + LA, bw)

            @pl.when(c < n_chunks)
            def _():
                wait_g(b)
                compute(b)
                start_s(c, b)

        for j in range(LA):
            start_g(j, j)

        def grp(i, _):
            c0 = NB * i
            for p in range(NB):
                phase(c0 + p, p)
            return 0
        lax.fori_loop(0, n_groups, grp, 0)

        for b in range(NB):
            wait_s(b)

    return k


@functools.lru_cache(maxsize=None)
def _get_kernel(V, D, B, S):
    return _make_sc_kernel(V, D, B, S)


def kernel(x, d_emb, data_table, pos_table):
    B, S = x.shape
    V, D = data_table.shape
    k = _get_kernel(V, D, B, S)
    return k(data_table, pos_table, x.astype(jnp.int32))
